# Initial kernel scaffold; baseline (speedup 1.0000x reference)
import functools

import jax
import jax.numpy as jnp
from jax import lax
from jax.experimental import pallas as pl
from jax.experimental.pallas import tpu as pltpu
from jax.experimental.pallas import tpu_sc as plsc

NC = 2
NS = 16
NW = NC * NS
CHUNK = 80


def _make_deg_kernel(n2, nch):
    stripe = n2 // NS
    mesh = plsc.VectorSubcoreMesh(core_axis_name="c", subcore_axis_name="s")

    @functools.partial(
        pl.kernel,
        out_type=jax.ShapeDtypeStruct((NC, n2, 16), jnp.float32),
        mesh=mesh,
        scratch_types=[
            pltpu.VMEM((nch, CHUNK), jnp.int32),
            pltpu.VMEM((CHUNK, 16), jnp.float32),
            pltpu.VMEM_SHARED((n2, 16), jnp.float32),
        ],
    )
    def deg_kernel(idx_hbm, ones_hbm, zeros_hbm, out_hbm, idx_v, ones_v, acc):
        cid = lax.axis_index("c")
        sid = lax.axis_index("s")
        wid = sid * NC + cid
        pltpu.sync_copy(zeros_hbm, acc.at[pl.ds(sid * stripe, stripe)])
        pltpu.sync_copy(idx_hbm.at[wid], idx_v)
        pltpu.sync_copy(ones_hbm, ones_v)
        plsc.subcore_barrier()

        def body(c, carry):
            pltpu.sync_copy(ones_v, acc.at[idx_v.at[c]], add=True)
            return carry

        lax.fori_loop(0, nch, body, 0)
        plsc.subcore_barrier()
        pltpu.sync_copy(
            acc.at[pl.ds(sid * stripe, stripe)],
            out_hbm.at[cid, pl.ds(sid * stripe, stripe)],
        )

    return deg_kernel


def _make_agg_kernel(n, d, nch):
    stripe = n // NS
    mesh = plsc.VectorSubcoreMesh(core_axis_name="c", subcore_axis_name="s")

    @functools.partial(
        pl.kernel,
        out_type=jax.ShapeDtypeStruct((NC, n, d), jnp.float32),
        mesh=mesh,
        scratch_types=[
            pltpu.VMEM((nch, CHUNK), jnp.int32),
            pltpu.VMEM((nch, CHUNK), jnp.int32),
            pltpu.VMEM((CHUNK, d), jnp.float32),
            pltpu.VMEM_SHARED((n, d), jnp.float32),
            pltpu.SemaphoreType.DMA,
        ],
    )
    def agg_kernel(h_hbm, src_hbm, dst_hbm, zeros_hbm, out_hbm,
                   sidx, didx, rows, acc, sem):
        cid = lax.axis_index("c")
        sid = lax.axis_index("s")
        wid = sid * NC + cid
        pltpu.sync_copy(zeros_hbm, acc.at[pl.ds(sid * stripe, stripe)])
        pltpu.sync_copy(src_hbm.at[wid], sidx)
        pltpu.sync_copy(dst_hbm.at[wid], didx)
        plsc.subcore_barrier()

        def body(c, carry):
            pltpu.async_copy(h_hbm.at[sidx.at[c]], rows, sem).wait()
            pltpu.sync_copy(rows, acc.at[didx.at[c]], add=True)
            return carry

        lax.fori_loop(0, nch, body, 0)
        plsc.subcore_barrier()
        pltpu.sync_copy(
            acc.at[pl.ds(sid * stripe, stripe)],
            out_hbm.at[cid, pl.ds(sid * stripe, stripe)],
        )

    return agg_kernel


def _mm1_body(x_ref, w_ref, do0, do1, di0, di1, h_ref, ns_ref, nd_ref):
    ns = lax.rsqrt(jnp.maximum(do0[...] + do1[...], 1.0))
    nd = lax.rsqrt(jnp.maximum(di0[...] + di1[...], 1.0))
    h_ref[...] = jnp.dot(x_ref[...], w_ref[...],
                         preferred_element_type=jnp.float32) * ns
    ns_ref[...] = ns
    nd_ref[...] = nd


def _mid_body(p0, p1, nd, ns, b, w, o_ref):
    t = (p0[...] + p1[...]) * nd[...] + b[...]
    t = jnp.maximum(t, 0.0)
    o_ref[...] = jnp.dot(t, w[...], preferred_element_type=jnp.float32) * ns[...]


def _fin_body(p0, p1, nd, b, o_ref):
    o_ref[...] = (p0[...] + p1[...]) * nd[...] + b[...]


def kernel(x, edge_index, W1, b1, W2, b2):
    n, d_in = x.shape
    d_hid = W1.shape[1]
    n_cls = W2.shape[1]
    e = edge_index.shape[1]
    assert e % (NW * CHUNK) == 0 and n % NS == 0

    ept = e // NW
    nch = ept // CHUNK
    src = edge_index[0]
    dst = edge_index[1]
    srcr = src.reshape(NW, nch, CHUNK)
    dstr = dst.reshape(NW, nch, CHUNK)
    degidx = jnp.concatenate([src, dst + n]).reshape(NW, 2 * nch, CHUNK)

    ones16 = jnp.ones((CHUNK, 16), jnp.float32)
    zeros_deg = jnp.zeros((2 * n // NS, 16), jnp.float32)
    zeros_h = jnp.zeros((n // NS, d_hid), jnp.float32)
    zeros_c = jnp.zeros((n // NS, n_cls), jnp.float32)

    degpart = _make_deg_kernel(2 * n, 2 * nch)(degidx, ones16, zeros_deg)
    do0 = degpart[0, :n, 0:1]
    do1 = degpart[1, :n, 0:1]
    di0 = degpart[0, n:, 0:1]
    di1 = degpart[1, n:, 0:1]

    bn = 1000
    grid = (n // bn,)
    h1p, ns_col, nd_col = pl.pallas_call(
        _mm1_body,
        grid=grid,
        in_specs=[
            pl.BlockSpec((bn, d_in), lambda i: (i, 0)),
            pl.BlockSpec((d_in, d_hid), lambda i: (0, 0)),
            pl.BlockSpec((bn, 1), lambda i: (i, 0)),
            pl.BlockSpec((bn, 1), lambda i: (i, 0)),
            pl.BlockSpec((bn, 1), lambda i: (i, 0)),
            pl.BlockSpec((bn, 1), lambda i: (i, 0)),
        ],
        out_specs=[
            pl.BlockSpec((bn, d_hid), lambda i: (i, 0)),
            pl.BlockSpec((bn, 1), lambda i: (i, 0)),
            pl.BlockSpec((bn, 1), lambda i: (i, 0)),
        ],
        out_shape=[
            jax.ShapeDtypeStruct((n, d_hid), jnp.float32),
            jax.ShapeDtypeStruct((n, 1), jnp.float32),
            jax.ShapeDtypeStruct((n, 1), jnp.float32),
        ],
    )(x, W1, do0, do1, di0, di1)

    part1 = _make_agg_kernel(n, d_hid, nch)(h1p, srcr, dstr, zeros_h)

    h2p = pl.pallas_call(
        _mid_body,
        grid=grid,
        in_specs=[
            pl.BlockSpec((bn, d_hid), lambda i: (i, 0)),
            pl.BlockSpec((bn, d_hid), lambda i: (i, 0)),
            pl.BlockSpec((bn, 1), lambda i: (i, 0)),
            pl.BlockSpec((bn, 1), lambda i: (i, 0)),
            pl.BlockSpec((1, d_hid), lambda i: (0, 0)),
            pl.BlockSpec((d_hid, n_cls), lambda i: (0, 0)),
        ],
        out_specs=pl.BlockSpec((bn, n_cls), lambda i: (i, 0)),
        out_shape=jax.ShapeDtypeStruct((n, n_cls), jnp.float32),
    )(part1[0], part1[1], nd_col, ns_col, b1.reshape(1, d_hid), W2)

    part2 = _make_agg_kernel(n, n_cls, nch)(h2p, srcr, dstr, zeros_c)

    out = pl.pallas_call(
        _fin_body,
        grid=grid,
        in_specs=[
            pl.BlockSpec((bn, n_cls), lambda i: (i, 0)),
            pl.BlockSpec((bn, n_cls), lambda i: (i, 0)),
            pl.BlockSpec((bn, 1), lambda i: (i, 0)),
            pl.BlockSpec((1, n_cls), lambda i: (0, 0)),
        ],
        out_specs=pl.BlockSpec((bn, n_cls), lambda i: (i, 0)),
        out_shape=jax.ShapeDtypeStruct((n, n_cls), jnp.float32),
    )(part2[0], part2[1], nd_col, b2.reshape(1, n_cls))

    return out


# speedup vs baseline: 6.6237x; 6.6237x over previous
import functools

import jax
import jax.numpy as jnp
from jax import lax
from jax.experimental import pallas as pl
from jax.experimental.pallas import tpu as pltpu
from jax.experimental.pallas import tpu_sc as plsc

NC = 2
NS = 16
NW = NC * NS
CHUNK = 80


def _make_deg_kernel(n2, ept2):
    mesh = plsc.VectorSubcoreMesh(core_axis_name="c", subcore_axis_name="s")

    @functools.partial(
        pl.kernel,
        out_type=jax.ShapeDtypeStruct((NC, NS, 1, n2), jnp.float32),
        mesh=mesh,
        scratch_types=[
            pltpu.VMEM((ept2,), jnp.int32),
            pltpu.VMEM((n2,), jnp.float32),
        ],
        compiler_params=pltpu.CompilerParams(needs_layout_passes=False),
    )
    def deg_kernel(idx_hbm, zeros_hbm, out_hbm, idx_v, hist):
        cid = lax.axis_index("c")
        sid = lax.axis_index("s")
        wid = sid * NC + cid
        pltpu.sync_copy(idx_hbm.at[wid, 0], idx_v)
        pltpu.sync_copy(zeros_hbm, hist)
        one16 = jnp.ones((16,), jnp.float32)

        def body(i, carry):
            vec = idx_v[pl.ds(pl.multiple_of(i * 16, 16), 16)]
            plsc.addupdate_scatter(hist, [vec], one16)
            return carry

        lax.fori_loop(0, ept2 // 16, body, 0)
        pltpu.sync_copy(hist, out_hbm.at[cid, sid, 0])

    return deg_kernel


def _make_agg_kernel(n, d, nch):
    stripe = n // NS
    mesh = plsc.VectorSubcoreMesh(core_axis_name="c", subcore_axis_name="s")

    @functools.partial(
        pl.kernel,
        out_type=jax.ShapeDtypeStruct((NC, n, d), jnp.float32),
        mesh=mesh,
        scratch_types=[
            pltpu.VMEM((nch, CHUNK), jnp.int32),
            pltpu.VMEM((nch, CHUNK), jnp.int32),
            pltpu.VMEM((CHUNK, d), jnp.float32),
            pltpu.VMEM_SHARED((n, d), jnp.float32),
            pltpu.SemaphoreType.DMA,
        ],
    )
    def agg_kernel(h_hbm, src_hbm, dst_hbm, zeros_hbm, out_hbm,
                   sidx, didx, rows, acc, sem):
        cid = lax.axis_index("c")
        sid = lax.axis_index("s")
        wid = sid * NC + cid
        pltpu.sync_copy(zeros_hbm, acc.at[pl.ds(sid * stripe, stripe)])
        pltpu.sync_copy(src_hbm.at[wid], sidx)
        pltpu.sync_copy(dst_hbm.at[wid], didx)
        plsc.subcore_barrier()

        def body(c, carry):
            pltpu.async_copy(h_hbm.at[sidx.at[c]], rows, sem).wait()
            pltpu.sync_copy(rows, acc.at[didx.at[c]], add=True)
            return carry

        lax.fori_loop(0, nch, body, 0)
        plsc.subcore_barrier()
        pltpu.sync_copy(
            acc.at[pl.ds(sid * stripe, stripe)],
            out_hbm.at[cid, pl.ds(sid * stripe, stripe)],
        )

    return agg_kernel


def _mm1_body(x_ref, w_ref, ds_ref, dd_ref, h_ref, ns_ref, nd_ref):
    ns = lax.rsqrt(jnp.maximum(
        jnp.sum(ds_ref[...], axis=1, keepdims=True), 1.0))
    nd = lax.rsqrt(jnp.maximum(
        jnp.sum(dd_ref[...], axis=1, keepdims=True), 1.0))
    h_ref[...] = jnp.dot(x_ref[...], w_ref[...],
                         preferred_element_type=jnp.float32) * ns
    ns_ref[...] = ns
    nd_ref[...] = nd


def _mid_body(p0, p1, nd, ns, b, o_ref):
    t = (p0[...] + p1[...]) * nd[...] + b[...]
    o_ref[...] = jnp.maximum(t, 0.0) * ns[...]


def _fin_body(p0, p1, nd, w, b, o_ref):
    s = p0[...] + p1[...]
    o_ref[...] = jnp.dot(s, w[...], preferred_element_type=jnp.float32) * nd[...] + b[...]


def kernel(x, edge_index, W1, b1, W2, b2):
    n, d_in = x.shape
    d_hid = W1.shape[1]
    n_cls = W2.shape[1]
    e = edge_index.shape[1]
    assert e % (NW * CHUNK) == 0 and n % NS == 0

    ept = e // NW
    nch = ept // CHUNK
    npad = -(-n // (8 * NS)) * (8 * NS)
    n2pad = -(-(2 * n) // 16) * 16
    src = edge_index[0]
    dst = edge_index[1]
    srcr = src.reshape(NW, nch, CHUNK)
    dstr = dst.reshape(NW, nch, CHUNK)
    degidx = jnp.concatenate([src, dst + n]).reshape(NW, 1, 2 * ept)

    zeros_h = jnp.zeros((npad // NS, d_hid), jnp.float32)

    degpart = _make_deg_kernel(n2pad, 2 * ept)(
        degidx, jnp.zeros((n2pad,), jnp.float32))
    deg_t = degpart.reshape(NW, n2pad).T

    bn = 1000
    noff = n // bn
    grid = (n // bn,)
    h1p, ns_col, nd_col = pl.pallas_call(
        _mm1_body,
        grid=grid,
        in_specs=[
            pl.BlockSpec((bn, d_in), lambda i: (i, 0)),
            pl.BlockSpec((d_in, d_hid), lambda i: (0, 0)),
            pl.BlockSpec((bn, NW), lambda i: (i, 0)),
            pl.BlockSpec((bn, NW), lambda i: (i + noff, 0)),
        ],
        out_specs=[
            pl.BlockSpec((bn, d_hid), lambda i: (i, 0)),
            pl.BlockSpec((bn, 1), lambda i: (i, 0)),
            pl.BlockSpec((bn, 1), lambda i: (i, 0)),
        ],
        out_shape=[
            jax.ShapeDtypeStruct((n, d_hid), jnp.float32),
            jax.ShapeDtypeStruct((n, 1), jnp.float32),
            jax.ShapeDtypeStruct((n, 1), jnp.float32),
        ],
    )(x, W1, deg_t, deg_t)

    part1 = _make_agg_kernel(npad, d_hid, nch)(h1p, srcr, dstr, zeros_h)

    h2p = pl.pallas_call(
        _mid_body,
        grid=grid,
        in_specs=[
            pl.BlockSpec((bn, d_hid), lambda i: (i, 0)),
            pl.BlockSpec((bn, d_hid), lambda i: (i, 0)),
            pl.BlockSpec((bn, 1), lambda i: (i, 0)),
            pl.BlockSpec((bn, 1), lambda i: (i, 0)),
            pl.BlockSpec((1, d_hid), lambda i: (0, 0)),
        ],
        out_specs=pl.BlockSpec((bn, d_hid), lambda i: (i, 0)),
        out_shape=jax.ShapeDtypeStruct((n, d_hid), jnp.float32),
    )(part1[0, :n], part1[1, :n], nd_col, ns_col, b1.reshape(1, d_hid))

    part2 = _make_agg_kernel(npad, d_hid, nch)(h2p, srcr, dstr, zeros_h)

    out = pl.pallas_call(
        _fin_body,
        grid=grid,
        in_specs=[
            pl.BlockSpec((bn, d_hid), lambda i: (i, 0)),
            pl.BlockSpec((bn, d_hid), lambda i: (i, 0)),
            pl.BlockSpec((bn, 1), lambda i: (i, 0)),
            pl.BlockSpec((d_hid, n_cls), lambda i: (0, 0)),
            pl.BlockSpec((1, n_cls), lambda i: (0, 0)),
        ],
        out_specs=pl.BlockSpec((bn, n_cls), lambda i: (i, 0)),
        out_shape=jax.ShapeDtypeStruct((n, n_cls), jnp.float32),
    )(part2[0, :n], part2[1, :n], nd_col, W2, b2.reshape(1, n_cls))

    return out
